# baseline (speedup 1.0000x reference)
import functools

import jax
import jax.numpy as jnp
from jax import lax
from jax.experimental import pallas as pl
from jax.experimental.pallas import tpu as pltpu
from jax.experimental.pallas import tpu_sc as plsc

_N = 10000
_D = 128
_H = 32
_E = 320000
_NC, _NS = 2, 16
_NW = _NC * _NS
_CSZ = 128
_CHUNKS = -(-_E // (_NW * _CSZ))
_EW = _CHUNKS * _CSZ
_EPAD = _NW * _EW
_RPT = 632
_AROWS = _NS * _RPT
_DEGW = 8
_NBUF = 8


def _sc_deg_body(dst_hbm, zero_hbm, ones_hbm, out_hbm, dst_v, ones_v, acc_sh):
    c = lax.axis_index("c")
    s = lax.axis_index("s")
    w = c * _NS + s
    pltpu.sync_copy(dst_hbm.at[w], dst_v)
    pltpu.sync_copy(ones_hbm, ones_v)

    @pl.when(s == 0)
    def _():
        pltpu.sync_copy(zero_hbm, acc_sh)

    plsc.subcore_barrier()

    def chunk(j, carry):
        pltpu.sync_copy(ones_v, acc_sh.at[dst_v.at[j]], add=True)
        return carry

    lax.fori_loop(0, _CHUNKS, chunk, 0)
    plsc.subcore_barrier()
    pltpu.sync_copy(acc_sh.at[pl.ds(s * _RPT, _RPT)],
                    out_hbm.at[c].at[pl.ds(s * _RPT, _RPT)])


def _sc_prop_body(xs_hbm, src_hbm, dst_hbm, zero_hbm, out_hbm,
                  src_v, dst_v, gbuf, acc_sh, gsem, ssem):
    c = lax.axis_index("c")
    s = lax.axis_index("s")
    w = c * _NS + s
    pltpu.sync_copy(src_hbm.at[w], src_v)
    pltpu.sync_copy(dst_hbm.at[w], dst_v)

    @pl.when(s == 0)
    def _():
        pltpu.sync_copy(zero_hbm, acc_sh)

    plsc.subcore_barrier()

    for p in range(_NBUF - 1):
        pltpu.async_copy(xs_hbm.at[c].at[src_v.at[p]], gbuf.at[p], gsem)

    def chunk(j, carry):
        pltpu.make_async_copy(xs_hbm.at[c].at[src_v.at[j]], gbuf.at[j % _NBUF],
                              gsem).wait()
        pltpu.async_copy(gbuf.at[j % _NBUF], acc_sh.at[dst_v.at[j]], ssem,
                         add=True)

        @pl.when(j + _NBUF - 1 < _CHUNKS)
        def _():
            @pl.when(j >= 1)
            def _():
                pltpu.make_async_copy(gbuf.at[j % _NBUF],
                                      acc_sh.at[dst_v.at[j]], ssem).wait()
            pltpu.async_copy(xs_hbm.at[c].at[src_v.at[j + _NBUF - 1]],
                             gbuf.at[(j + _NBUF - 1) % _NBUF], gsem)

        return carry

    lax.fori_loop(0, _CHUNKS, chunk, 0)
    for p in range(_NBUF):
        pltpu.make_async_copy(gbuf.at[p], acc_sh.at[dst_v.at[p]], ssem).wait()
    plsc.subcore_barrier()
    pltpu.sync_copy(acc_sh.at[pl.ds(s * _RPT, _RPT)],
                    out_hbm.at[c].at[pl.ds(s * _RPT, _RPT)])


@functools.lru_cache(maxsize=None)
def _sc_kernels():
    mesh = plsc.VectorSubcoreMesh(core_axis_name="c", subcore_axis_name="s",
                                  num_cores=_NC, num_subcores=_NS)
    params = pltpu.CompilerParams(use_tc_tiling_on_sc=False)
    sc_deg = pl.kernel(
        _sc_deg_body,
        out_type=jax.ShapeDtypeStruct((_NC, _AROWS, _DEGW), jnp.float32),
        mesh=mesh,
        scratch_types=[
            pltpu.VMEM((_CHUNKS, _CSZ), jnp.int32),
            pltpu.VMEM((_CSZ, _DEGW), jnp.float32),
            pltpu.VMEM_SHARED((_AROWS, _DEGW), jnp.float32),
        ],
        compiler_params=params,
    )
    sc_prop = pl.kernel(
        _sc_prop_body,
        out_type=jax.ShapeDtypeStruct((_NC, _AROWS, _H), jnp.float32),
        mesh=mesh,
        scratch_types=[
            pltpu.VMEM((_CHUNKS, _CSZ), jnp.int32),
            pltpu.VMEM((_CHUNKS, _CSZ), jnp.int32),
            pltpu.VMEM((_NBUF, _CSZ, _H), jnp.float32),
            pltpu.VMEM_SHARED((_AROWS, _H), jnp.float32),
            pltpu.SemaphoreType.DMA,
            pltpu.SemaphoreType.DMA,
        ],
        compiler_params=params,
    )
    return sc_deg, sc_prop


def _tc1_body(x_ref, w1_ref, deg_ref, xs_ref, dis_ref):
    h = jnp.dot(x_ref[...], w1_ref[...], preferred_element_type=jnp.float32)
    d = deg_ref[0, 0:_N, 0:1] + deg_ref[1, 0:_N, 0:1] + 1.0
    dis = lax.rsqrt(d)
    xs_ref[...] = jnp.broadcast_to((h * dis)[None], (_NC, _N, _H))
    dis_ref[...] = dis


_tc1 = pl.pallas_call(
    _tc1_body,
    out_shape=[jax.ShapeDtypeStruct((_NC, _N, _H), jnp.float32),
               jax.ShapeDtypeStruct((_N, 1), jnp.float32)],
)


def _tc2_body(acc_ref, xs_ref, dis_ref, b_ref, w_ref, out_ref):
    a = acc_ref[0, 0:_N] + acc_ref[1, 0:_N] + xs_ref[0]
    h = jnp.maximum(dis_ref[...] * a + b_ref[...], 0.0)
    hw = jnp.dot(h, w_ref[...],
                 preferred_element_type=jnp.float32) * dis_ref[...]
    out_ref[...] = jnp.broadcast_to(hw[None], (_NC, _N, _H))


_tc2 = pl.pallas_call(
    _tc2_body,
    out_shape=jax.ShapeDtypeStruct((_NC, _N, _H), jnp.float32),
)


def _tc3_body(acc_ref, xs_ref, dis_ref, b_ref, w_ref, fcb_ref, out_ref):
    a = acc_ref[0, 0:_N] + acc_ref[1, 0:_N] + xs_ref[0]
    h = jnp.maximum(dis_ref[...] * a + b_ref[...], 0.0)
    out_ref[...] = jnp.dot(h, w_ref[...],
                           preferred_element_type=jnp.float32) + fcb_ref[...]


_tc3 = pl.pallas_call(
    _tc3_body,
    out_shape=jax.ShapeDtypeStruct((_N, 1), jnp.float32),
)


def kernel(x, edge_index, W1, b1, W2, b2, fc_W, fc_b):
    pad = _EPAD - _E
    pad_dst = _N + jnp.arange(pad, dtype=jnp.int32) % (_AROWS - _N)
    srcp = jnp.concatenate(
        [edge_index[0], jnp.zeros((pad,), jnp.int32)]).reshape(_NW, _CHUNKS, _CSZ)
    dstp = jnp.concatenate(
        [edge_index[1], pad_dst]).reshape(_NW, _CHUNKS, _CSZ)
    zeros_deg = jnp.zeros((_AROWS, _DEGW), jnp.float32)
    ones_deg = jnp.ones((_CSZ, _DEGW), jnp.float32)
    zeros_acc = jnp.zeros((_AROWS, _H), jnp.float32)

    _sc_deg, _sc_prop = _sc_kernels()
    deg2 = _sc_deg(dstp, zeros_deg, ones_deg)
    xs1, dis = _tc1(x, W1, deg2)
    acc1 = _sc_prop(xs1, srcp, dstp, zeros_acc)
    xs2 = _tc2(acc1, xs1, dis, b1.reshape(1, _H), W2)
    acc2 = _sc_prop(xs2, srcp, dstp, zeros_acc)
    score = _tc3(acc2, xs2, dis, b2.reshape(1, _H), fc_W, fc_b.reshape(1, 1))
    return score.reshape(_N)


# speedup vs baseline: 1.0412x; 1.0412x over previous
import functools

import jax
import jax.numpy as jnp
from jax import lax
from jax.experimental import pallas as pl
from jax.experimental.pallas import tpu as pltpu
from jax.experimental.pallas import tpu_sc as plsc

_N = 10000
_D = 128
_H = 32
_E = 320000
_NC, _NS = 2, 16
_NW = _NC * _NS
_CSZ = 128
_CHUNKS = -(-_E // (_NW * _CSZ))
_EW = _CHUNKS * _CSZ
_EPAD = _NW * _EW
_RPT = 632
_AROWS = _NS * _RPT
_DEGW = 8
_NBUF = 8


def _sc_deg_body(dst_hbm, zero_hbm, ones_hbm, out_hbm, dst_v, ones_v, acc_sh):
    c = lax.axis_index("c")
    s = lax.axis_index("s")
    w = c * _NS + s
    pltpu.sync_copy(dst_hbm.at[w], dst_v)
    pltpu.sync_copy(ones_hbm, ones_v)

    @pl.when(s == 0)
    def _():
        pltpu.sync_copy(zero_hbm, acc_sh)

    plsc.subcore_barrier()

    def chunk(j, carry):
        pltpu.sync_copy(ones_v, acc_sh.at[dst_v.at[j]], add=True)
        return carry

    lax.fori_loop(0, _CHUNKS, chunk, 0)
    plsc.subcore_barrier()
    pltpu.sync_copy(acc_sh.at[pl.ds(s * _RPT, _RPT)],
                    out_hbm.at[c].at[pl.ds(s * _RPT, _RPT)])


def _sc_prop_body(xs_hbm, src_hbm, dst_hbm, zero_hbm, out_hbm,
                  src_v, dst_v, gbuf, acc_sh, gsem, ssem):
    c = lax.axis_index("c")
    s = lax.axis_index("s")
    w = c * _NS + s
    pltpu.sync_copy(src_hbm.at[w], src_v)
    pltpu.sync_copy(dst_hbm.at[w], dst_v)

    @pl.when(s == 0)
    def _():
        pltpu.sync_copy(zero_hbm, acc_sh)

    plsc.subcore_barrier()

    for p in range(_NBUF - 1):
        pltpu.async_copy(xs_hbm.at[src_v.at[p]], gbuf.at[p], gsem)

    def chunk(j, carry):
        pltpu.make_async_copy(xs_hbm.at[src_v.at[j]], gbuf.at[j % _NBUF],
                              gsem).wait()
        pltpu.async_copy(gbuf.at[j % _NBUF], acc_sh.at[dst_v.at[j]], ssem,
                         add=True)

        @pl.when(j + _NBUF - 1 < _CHUNKS)
        def _():
            @pl.when(j >= 1)
            def _():
                pltpu.make_async_copy(gbuf.at[j % _NBUF],
                                      acc_sh.at[dst_v.at[j]], ssem).wait()
            pltpu.async_copy(xs_hbm.at[src_v.at[j + _NBUF - 1]],
                             gbuf.at[(j + _NBUF - 1) % _NBUF], gsem)

        return carry

    lax.fori_loop(0, _CHUNKS, chunk, 0)
    for p in range(_NBUF):
        pltpu.make_async_copy(gbuf.at[p], acc_sh.at[dst_v.at[p]], ssem).wait()
    plsc.subcore_barrier()
    pltpu.sync_copy(acc_sh.at[pl.ds(s * _RPT, _RPT)],
                    out_hbm.at[c].at[pl.ds(s * _RPT, _RPT)])


@functools.lru_cache(maxsize=None)
def _sc_kernels():
    mesh = plsc.VectorSubcoreMesh(core_axis_name="c", subcore_axis_name="s",
                                  num_cores=_NC, num_subcores=_NS)
    params = pltpu.CompilerParams(use_tc_tiling_on_sc=False)
    sc_deg = pl.kernel(
        _sc_deg_body,
        out_type=jax.ShapeDtypeStruct((_NC, _AROWS, _DEGW), jnp.float32),
        mesh=mesh,
        scratch_types=[
            pltpu.VMEM((_CHUNKS, _CSZ), jnp.int32),
            pltpu.VMEM((_CSZ, _DEGW), jnp.float32),
            pltpu.VMEM_SHARED((_AROWS, _DEGW), jnp.float32),
        ],
        compiler_params=params,
    )
    sc_prop = pl.kernel(
        _sc_prop_body,
        out_type=jax.ShapeDtypeStruct((_NC, _AROWS, _H), jnp.float32),
        mesh=mesh,
        scratch_types=[
            pltpu.VMEM((_CHUNKS, _CSZ), jnp.int32),
            pltpu.VMEM((_CHUNKS, _CSZ), jnp.int32),
            pltpu.VMEM((_NBUF, _CSZ, _H), jnp.float32),
            pltpu.VMEM_SHARED((_AROWS, _H), jnp.float32),
            pltpu.SemaphoreType.DMA,
            pltpu.SemaphoreType.DMA,
        ],
        compiler_params=params,
    )
    return sc_deg, sc_prop


def _tc0_body(x_ref, w1_ref, h_ref):
    h_ref[...] = jnp.dot(x_ref[...], w1_ref[...],
                         preferred_element_type=jnp.float32)


_tc0 = pl.pallas_call(
    _tc0_body,
    out_shape=jax.ShapeDtypeStruct((_N, _H), jnp.float32),
)


def _tc1_body(h_ref, deg_ref, xs_ref, dis_ref):
    d = deg_ref[0, 0:_N, 0:1] + deg_ref[1, 0:_N, 0:1] + 1.0
    dis = lax.rsqrt(d)
    xs_ref[...] = h_ref[...] * dis
    dis_ref[...] = dis


_tc1 = pl.pallas_call(
    _tc1_body,
    out_shape=[jax.ShapeDtypeStruct((_N, _H), jnp.float32),
               jax.ShapeDtypeStruct((_N, 1), jnp.float32)],
)


def _tc2_body(acc_ref, xs_ref, dis_ref, b_ref, w_ref, out_ref):
    a = acc_ref[0, 0:_N] + acc_ref[1, 0:_N] + xs_ref[...]
    h = jnp.maximum(dis_ref[...] * a + b_ref[...], 0.0)
    out_ref[...] = jnp.dot(h, w_ref[...],
                           preferred_element_type=jnp.float32) * dis_ref[...]


_tc2 = pl.pallas_call(
    _tc2_body,
    out_shape=jax.ShapeDtypeStruct((_N, _H), jnp.float32),
)


def _tc3_body(acc_ref, xs_ref, dis_ref, b_ref, w_ref, fcb_ref, out_ref):
    a = acc_ref[0, 0:_N] + acc_ref[1, 0:_N] + xs_ref[...]
    h = jnp.maximum(dis_ref[...] * a + b_ref[...], 0.0)
    out_ref[...] = jnp.dot(h, w_ref[...],
                           preferred_element_type=jnp.float32) + fcb_ref[...]


_tc3 = pl.pallas_call(
    _tc3_body,
    out_shape=jax.ShapeDtypeStruct((_N, 1), jnp.float32),
)


def kernel(x, edge_index, W1, b1, W2, b2, fc_W, fc_b):
    pad = _EPAD - _E
    pad_dst = _N + jnp.arange(pad, dtype=jnp.int32) % (_AROWS - _N)
    srcp = jnp.concatenate(
        [edge_index[0], jnp.zeros((pad,), jnp.int32)]).reshape(_NW, _CHUNKS, _CSZ)
    dstp = jnp.concatenate(
        [edge_index[1], pad_dst]).reshape(_NW, _CHUNKS, _CSZ)
    zeros_deg = jnp.zeros((_AROWS, _DEGW), jnp.float32)
    ones_deg = jnp.ones((_CSZ, _DEGW), jnp.float32)
    zeros_acc = jnp.zeros((_AROWS, _H), jnp.float32)

    _sc_deg, _sc_prop = _sc_kernels()
    deg2 = _sc_deg(dstp, zeros_deg, ones_deg)
    h1 = _tc0(x, W1)
    xs1, dis = _tc1(h1, deg2)
    acc1 = _sc_prop(xs1, srcp, dstp, zeros_acc)
    xs2 = _tc2(acc1, xs1, dis, b1.reshape(1, _H), W2)
    acc2 = _sc_prop(xs2, srcp, dstp, zeros_acc)
    score = _tc3(acc2, xs2, dis, b2.reshape(1, _H), fc_W, fc_b.reshape(1, 1))
    return score.reshape(_N)


# speedup vs baseline: 1.0756x; 1.0330x over previous
import functools

import jax
import jax.numpy as jnp
from jax import lax
from jax.experimental import pallas as pl
from jax.experimental.pallas import tpu as pltpu
from jax.experimental.pallas import tpu_sc as plsc

_N = 10000
_D = 128
_H = 32
_E = 320000
_NC, _NS = 2, 16
_NW = _NC * _NS
_CSZ = 128
_CHUNKS = -(-_E // (_NW * _CSZ))
_EW = _CHUNKS * _CSZ
_EPAD = _NW * _EW
_RPT = 632
_AROWS = _NS * _RPT
_DEGW = 8
_NBUF = 8


def _sc_deg_body(dst_hbm, zero_hbm, ones_hbm, out_hbm, dst_v, ones_v, acc_sh):
    c = lax.axis_index("c")
    s = lax.axis_index("s")
    w = c * _NS + s
    pltpu.sync_copy(dst_hbm.at[w], dst_v)
    pltpu.sync_copy(ones_hbm, ones_v)

    @pl.when(s == 0)
    def _():
        pltpu.sync_copy(zero_hbm, acc_sh)

    plsc.subcore_barrier()

    def chunk(j, carry):
        pltpu.sync_copy(ones_v, acc_sh.at[dst_v.at[j]], add=True)
        return carry

    lax.fori_loop(0, _CHUNKS, chunk, 0)
    plsc.subcore_barrier()
    pltpu.sync_copy(acc_sh.at[pl.ds(s * _RPT, _RPT)],
                    out_hbm.at[c].at[pl.ds(s * _RPT, _RPT)])


def _sc_prop_body(xs_hbm, src_hbm, dst_hbm, zero_hbm, out_hbm,
                  src_v, dst_v, gbuf, acc_sh, gsem, ssem):
    c = lax.axis_index("c")
    s = lax.axis_index("s")
    w = c * _NS + s
    pltpu.sync_copy(src_hbm.at[w], src_v)
    pltpu.sync_copy(dst_hbm.at[w], dst_v)

    @pl.when(s == 0)
    def _():
        pltpu.sync_copy(zero_hbm, acc_sh)

    plsc.subcore_barrier()

    for p in range(_NBUF - 1):
        pltpu.async_copy(xs_hbm.at[src_v.at[p]], gbuf.at[p], gsem)

    def chunk(j, carry):
        pltpu.make_async_copy(xs_hbm.at[src_v.at[j]], gbuf.at[j % _NBUF],
                              gsem).wait()
        pltpu.async_copy(gbuf.at[j % _NBUF], acc_sh.at[dst_v.at[j]], ssem,
                         add=True)

        @pl.when(j + _NBUF - 1 < _CHUNKS)
        def _():
            @pl.when(j >= 1)
            def _():
                pltpu.make_async_copy(gbuf.at[j % _NBUF],
                                      acc_sh.at[dst_v.at[j]], ssem).wait()
            pltpu.async_copy(xs_hbm.at[src_v.at[j + _NBUF - 1]],
                             gbuf.at[(j + _NBUF - 1) % _NBUF], gsem)

        return carry

    lax.fori_loop(0, _CHUNKS, chunk, 0)
    for p in range(_NBUF):
        pltpu.make_async_copy(gbuf.at[p], acc_sh.at[dst_v.at[p]], ssem).wait()
    plsc.subcore_barrier()
    pltpu.sync_copy(acc_sh.at[pl.ds(s * _RPT, _RPT)],
                    out_hbm.at[c].at[pl.ds(s * _RPT, _RPT)])


@functools.lru_cache(maxsize=None)
def _sc_kernels():
    mesh = plsc.VectorSubcoreMesh(core_axis_name="c", subcore_axis_name="s",
                                  num_cores=_NC, num_subcores=_NS)
    params = pltpu.CompilerParams(use_tc_tiling_on_sc=False)
    sc_deg = pl.kernel(
        _sc_deg_body,
        out_type=jax.ShapeDtypeStruct((_NC, _AROWS, _DEGW), jnp.float32),
        mesh=mesh,
        scratch_types=[
            pltpu.VMEM((_CHUNKS, _CSZ), jnp.int32),
            pltpu.VMEM((_CSZ, _DEGW), jnp.float32),
            pltpu.VMEM_SHARED((_AROWS, _DEGW), jnp.float32),
        ],
        compiler_params=params,
    )
    sc_prop = pl.kernel(
        _sc_prop_body,
        out_type=jax.ShapeDtypeStruct((_NC, _AROWS, _H), jnp.float32),
        mesh=mesh,
        scratch_types=[
            pltpu.VMEM((_CHUNKS, _CSZ), jnp.int32),
            pltpu.VMEM((_CHUNKS, _CSZ), jnp.int32),
            pltpu.VMEM((_NBUF, _CSZ, _H), jnp.float32),
            pltpu.VMEM_SHARED((_AROWS, _H), jnp.float32),
            pltpu.SemaphoreType.DMA,
            pltpu.SemaphoreType.DMA,
        ],
        compiler_params=params,
    )
    return sc_deg, sc_prop


def _tc1_body(x_ref, w1_ref, deg_ref, xs_ref, dis_ref):
    h = jnp.dot(x_ref[...], w1_ref[...], preferred_element_type=jnp.float32)
    d = deg_ref[0, 0:_N, 0:1] + deg_ref[1, 0:_N, 0:1] + 1.0
    dis = lax.rsqrt(d)
    xs_ref[...] = h * dis
    dis_ref[...] = dis


_tc1 = pl.pallas_call(
    _tc1_body,
    out_shape=[jax.ShapeDtypeStruct((_N, _H), jnp.float32),
               jax.ShapeDtypeStruct((_N, 1), jnp.float32)],
)


def _tc2_body(acc_ref, xs_ref, dis_ref, b_ref, w_ref, out_ref):
    a = acc_ref[0, 0:_N] + acc_ref[1, 0:_N] + xs_ref[...]
    h = jnp.maximum(dis_ref[...] * a + b_ref[...], 0.0)
    out_ref[...] = jnp.dot(h, w_ref[...],
                           preferred_element_type=jnp.float32) * dis_ref[...]


_tc2 = pl.pallas_call(
    _tc2_body,
    out_shape=jax.ShapeDtypeStruct((_N, _H), jnp.float32),
)


def _tc3_body(acc_ref, xs_ref, dis_ref, b_ref, w_ref, fcb_ref, out_ref):
    a = acc_ref[0, 0:_N] + acc_ref[1, 0:_N] + xs_ref[...]
    h = jnp.maximum(dis_ref[...] * a + b_ref[...], 0.0)
    out_ref[...] = jnp.dot(h, w_ref[...],
                           preferred_element_type=jnp.float32) + fcb_ref[...]


_tc3 = pl.pallas_call(
    _tc3_body,
    out_shape=jax.ShapeDtypeStruct((_N, 1), jnp.float32),
)


def kernel(x, edge_index, W1, b1, W2, b2, fc_W, fc_b):
    pad = _EPAD - _E
    pad_dst = _N + jnp.arange(pad, dtype=jnp.int32) % (_AROWS - _N)
    srcp = jnp.concatenate(
        [edge_index[0], jnp.zeros((pad,), jnp.int32)]).reshape(_NW, _CHUNKS, _CSZ)
    dstp = jnp.concatenate(
        [edge_index[1], pad_dst]).reshape(_NW, _CHUNKS, _CSZ)
    zeros_deg = jnp.zeros((_AROWS, _DEGW), jnp.float32)
    ones_deg = jnp.ones((_CSZ, _DEGW), jnp.float32)
    zeros_acc = jnp.zeros((_AROWS, _H), jnp.float32)

    _sc_deg, _sc_prop = _sc_kernels()
    deg2 = _sc_deg(dstp, zeros_deg, ones_deg)
    xs1, dis = _tc1(x, W1, deg2)
    acc1 = _sc_prop(xs1, srcp, dstp, zeros_acc)
    xs2 = _tc2(acc1, xs1, dis, b1.reshape(1, _H), W2)
    acc2 = _sc_prop(xs2, srcp, dstp, zeros_acc)
    score = _tc3(acc2, xs2, dis, b2.reshape(1, _H), fc_W, fc_b.reshape(1, 1))
    return score.reshape(_N)


# speedup vs baseline: 1.1026x; 1.0251x over previous
import functools

import jax
import jax.numpy as jnp
from jax import lax
from jax.experimental import pallas as pl
from jax.experimental.pallas import tpu as pltpu
from jax.experimental.pallas import tpu_sc as plsc

_N = 10000
_D = 128
_H = 32
_E = 320000
_NC, _NS = 2, 16
_NW = _NC * _NS
_CSZ = 128
_CHUNKS = -(-_E // (_NW * _CSZ))
_EW = _CHUNKS * _CSZ
_EPAD = _NW * _EW
_RPT = 632
_AROWS = _NS * _RPT
_DEGW = 8
_NBUF = 8


def _sc_deg_body(dst_hbm, zero_hbm, ones_hbm, out_hbm, dst_v, ones_v, acc_sh):
    c = lax.axis_index("c")
    s = lax.axis_index("s")
    w = c * _NS + s
    pltpu.sync_copy(dst_hbm.at[w], dst_v)
    pltpu.sync_copy(ones_hbm, ones_v)

    @pl.when(s == 0)
    def _():
        pltpu.sync_copy(zero_hbm, acc_sh)

    plsc.subcore_barrier()

    def chunk(j, carry):
        pltpu.sync_copy(ones_v, acc_sh.at[dst_v.at[j]], add=True)
        return carry

    lax.fori_loop(0, _CHUNKS, chunk, 0)
    plsc.subcore_barrier()
    pltpu.sync_copy(acc_sh.at[pl.ds(s * _RPT, _RPT)],
                    out_hbm.at[c].at[pl.ds(s * _RPT, _RPT)])


def _sc_prop_body(xs_hbm, src_hbm, dst_hbm, zero_hbm, out_hbm,
                  src_v, dst_v, gbuf, acc_sh, gsem, ssem):
    c = lax.axis_index("c")
    s = lax.axis_index("s")
    w = c * _NS + s
    pltpu.sync_copy(src_hbm.at[w], src_v)
    pltpu.sync_copy(dst_hbm.at[w], dst_v)

    @pl.when(s == 0)
    def _():
        pltpu.sync_copy(zero_hbm, acc_sh)

    plsc.subcore_barrier()

    for p in range(_NBUF - 1):
        pltpu.async_copy(xs_hbm.at[src_v.at[p]], gbuf.at[p], gsem)

    def chunk(j, carry):
        pltpu.make_async_copy(xs_hbm.at[src_v.at[j]], gbuf.at[j % _NBUF],
                              gsem).wait()
        pltpu.async_copy(gbuf.at[j % _NBUF], acc_sh.at[dst_v.at[j]], ssem,
                         add=True)

        @pl.when(j + _NBUF - 1 < _CHUNKS)
        def _():
            @pl.when(j >= 1)
            def _():
                pltpu.make_async_copy(gbuf.at[j % _NBUF],
                                      acc_sh.at[dst_v.at[j]], ssem).wait()
            pltpu.async_copy(xs_hbm.at[src_v.at[j + _NBUF - 1]],
                             gbuf.at[(j + _NBUF - 1) % _NBUF], gsem)

        return carry

    lax.fori_loop(0, _CHUNKS, chunk, 0)
    for p in range(_NBUF):
        pltpu.make_async_copy(gbuf.at[p], acc_sh.at[dst_v.at[p]], ssem).wait()
    plsc.subcore_barrier()
    pltpu.sync_copy(acc_sh.at[pl.ds(s * _RPT, _RPT)],
                    out_hbm.at[c].at[pl.ds(s * _RPT, _RPT)])


@functools.lru_cache(maxsize=None)
def _sc_kernels():
    mesh = plsc.VectorSubcoreMesh(core_axis_name="c", subcore_axis_name="s",
                                  num_cores=_NC, num_subcores=_NS)
    params = pltpu.CompilerParams(use_tc_tiling_on_sc=False)
    sc_deg = pl.kernel(
        _sc_deg_body,
        out_type=jax.ShapeDtypeStruct((_NC, _AROWS, _DEGW), jnp.float32),
        mesh=mesh,
        scratch_types=[
            pltpu.VMEM((_CHUNKS, _CSZ), jnp.int32),
            pltpu.VMEM((_CSZ, _DEGW), jnp.float32),
            pltpu.VMEM_SHARED((_AROWS, _DEGW), jnp.float32),
        ],
        compiler_params=params,
    )
    sc_prop = pl.kernel(
        _sc_prop_body,
        out_type=jax.ShapeDtypeStruct((_NC, _AROWS, _H), jnp.float32),
        mesh=mesh,
        scratch_types=[
            pltpu.VMEM((_CHUNKS, _CSZ), jnp.int32),
            pltpu.VMEM((_CHUNKS, _CSZ), jnp.int32),
            pltpu.VMEM((_NBUF, _CSZ, _H), jnp.float32),
            pltpu.VMEM_SHARED((_AROWS, _H), jnp.float32),
            pltpu.SemaphoreType.DMA,
            pltpu.SemaphoreType.DMA,
        ],
        compiler_params=params,
    )
    return sc_deg, sc_prop


def _tc1_body(x_ref, w1_ref, deg_ref, xs_ref, dis_ref):
    h = jnp.dot(x_ref[...], w1_ref[...], preferred_element_type=jnp.float32)
    d = deg_ref[0, 0:_N, 0:1] + deg_ref[1, 0:_N, 0:1] + 1.0
    dis = lax.rsqrt(d)
    xs_ref[...] = h * dis
    dis_ref[...] = dis


_tc1 = pl.pallas_call(
    _tc1_body,
    out_shape=[jax.ShapeDtypeStruct((_N, _H), jnp.float32),
               jax.ShapeDtypeStruct((_N, 1), jnp.float32)],
)


def _tc2_body(acc_ref, xs_ref, dis_ref, b_ref, w_ref, out_ref):
    a = acc_ref[0, 0:_N] + acc_ref[1, 0:_N] + xs_ref[...]
    h = jnp.maximum(dis_ref[...] * a + b_ref[...], 0.0)
    out_ref[...] = jnp.dot(h, w_ref[...],
                           preferred_element_type=jnp.float32) * dis_ref[...]


_tc2 = pl.pallas_call(
    _tc2_body,
    out_shape=jax.ShapeDtypeStruct((_N, _H), jnp.float32),
)


def _tc3_body(acc_ref, xs_ref, dis_ref, b_ref, w_ref, fcb_ref, out_ref):
    a = acc_ref[0, 0:_N] + acc_ref[1, 0:_N] + xs_ref[...]
    h = jnp.maximum(dis_ref[...] * a + b_ref[...], 0.0)
    s = jnp.dot(h, w_ref[...],
                preferred_element_type=jnp.float32) + fcb_ref[...]
    out_ref[...] = s[:, 0]


_tc3 = pl.pallas_call(
    _tc3_body,
    out_shape=jax.ShapeDtypeStruct((_N,), jnp.float32),
)


def kernel(x, edge_index, W1, b1, W2, b2, fc_W, fc_b):
    pad = _EPAD - _E
    pad_dst = _N + jnp.arange(pad, dtype=jnp.int32) % (_AROWS - _N)
    srcp = jnp.concatenate(
        [edge_index[0], jnp.zeros((pad,), jnp.int32)]).reshape(_NW, _CHUNKS, _CSZ)
    dstp = jnp.concatenate(
        [edge_index[1], pad_dst]).reshape(_NW, _CHUNKS, _CSZ)
    zeros_deg = jnp.zeros((_AROWS, _DEGW), jnp.float32)
    ones_deg = jnp.ones((_CSZ, _DEGW), jnp.float32)
    zeros_acc = jnp.zeros((_AROWS, _H), jnp.float32)

    _sc_deg, _sc_prop = _sc_kernels()
    deg2 = _sc_deg(dstp, zeros_deg, ones_deg)
    xs1, dis = _tc1(x, W1, deg2)
    acc1 = _sc_prop(xs1, srcp, dstp, zeros_acc)
    xs2 = _tc2(acc1, xs1, dis, b1.reshape(1, _H), W2)
    acc2 = _sc_prop(xs2, srcp, dstp, zeros_acc)
    return _tc3(acc2, xs2, dis, b2.reshape(1, _H), fc_W, fc_b.reshape(1, 1))
